# baseline (speedup 1.0000x reference)
import functools

import numpy as np
import jax
import jax.numpy as jnp
from jax import lax
from jax.experimental import pallas as pl
from jax.experimental.pallas import tpu as pltpu
from jax.experimental.pallas import tpu_sc as plsc

_H = 8
_C = 16
_ACCW = 144

_PERM = np.empty(128, np.int32)
for _k in range(8):
    for _j in range(16):
        _PERM[16 * _k + _j] = (_j % 8) * 16 + (2 * _k + _j // 8)

_PM = np.zeros((128, 128), np.float32)
for _i in range(128):
    _PM[_i, _PERM[_i]] = 1.0

_RT = np.zeros((8, 128), np.float32)
for _j in range(128):
    _RT[_PERM[_j] // 16, _j] = 1.0

_S16 = np.zeros((128, 16), np.float32)
for _j in range(128):
    _S16[_j, _PERM[_j] // 16] = 1.0
    _S16[_j, 8 + _PERM[_j] // 16] = 1.0


def _tc_pre_body(x_ref, g1_ref, b1_ref, wp_ref, ast_ref, adt_ref, rwp_ref,
                 s16_ref, xp_ref, as2_ref, ad2_ref, sw_ref, res_ref):
    xb = x_ref[...]
    mu = jnp.mean(xb, axis=-1, keepdims=True)
    xc = xb - mu
    var = jnp.mean(xc * xc, axis=-1, keepdims=True)
    h = xc / jnp.sqrt(var + 1e-5) * g1_ref[...] + b1_ref[...]
    xp_t = jnp.dot(h, wp_ref[...], preferred_element_type=jnp.float32)
    as2 = jnp.dot(xp_t * ast_ref[...], s16_ref[...],
                  preferred_element_type=jnp.float32)
    ad2 = jnp.dot(xp_t * adt_ref[...], s16_ref[...],
                  preferred_element_type=jnp.float32)
    res_t = jnp.dot(h, rwp_ref[...], preferred_element_type=jnp.float32)
    z = as2[:, :8] + ad2[:, :8]
    w = jnp.exp(jnp.maximum(z, 0.2 * z))
    xp_ref[...] = xp_t
    as2_ref[...] = as2
    ad2_ref[...] = ad2
    sw_ref[...] = w
    res_ref[...] = res_t


def _tc_post_body(o0_ref, o1_ref, d0_ref, d1_ref, xp_ref, sw_ref, res_ref,
                  bp_ref, g2_ref, b2_ref, wf1_ref, bf1_ref, wf2_ref, bf2_ref,
                  pm_ref, rt_ref, y_ref):
    sw = sw_ref[...]
    swrep = jnp.dot(sw, rt_ref[...], preferred_element_type=jnp.float32)
    num = o0_ref[...] + o1_ref[...] + xp_ref[...] * swrep
    den = (d0_ref[...] + d1_ref[...])[:, :8] + sw
    den_rep = jnp.dot(den, rt_ref[...], preferred_element_type=jnp.float32)
    g = num / (den_rep + 1e-16) + res_ref[...] + bp_ref[...]
    mu = jnp.mean(g, axis=-1, keepdims=True)
    gc = g - mu
    var = jnp.mean(gc * gc, axis=-1, keepdims=True)
    h2 = gc / jnp.sqrt(var + 1e-5) * g2_ref[...] + b2_ref[...]
    u = jnp.dot(h2, wf1_ref[...], preferred_element_type=jnp.float32)
    u = u + bf1_ref[...]
    ff = 0.5 * u * (1.0 + jnp.tanh(np.sqrt(2.0 / np.pi)
                                   * (u + 0.044715 * u * u * u)))
    y = jnp.dot(h2, pm_ref[...], preferred_element_type=jnp.float32)
    y = y + jnp.dot(ff, wf2_ref[...], preferred_element_type=jnp.float32)
    y_ref[...] = y + bf2_ref[...]


def _full(shape):
    return pl.BlockSpec(shape, lambda i: (0,) * len(shape))


def _rows(rb, w):
    return pl.BlockSpec((rb, w), lambda i: (i, 0))


def _make_sc_edge_call(n, e):
    nc, ns = 2, 16
    k = 40
    nbuf = 3
    sbc = 50
    e_per_core = e // nc
    e_per_tile = e_per_core // ns
    n_edge_chunks = e_per_tile // k
    nsb = n_edge_chunks // sbc
    assert e_per_tile % k == 0 and n % k == 0 and n_edge_chunks % sbc == 0
    n_row_chunks = n // k
    mesh = plsc.VectorSubcoreMesh(core_axis_name="c", subcore_axis_name="s")

    @functools.partial(
        pl.kernel,
        mesh=mesh,
        compiler_params=pltpu.CompilerParams(use_tc_tiling_on_sc=False),
        out_type=(jax.ShapeDtypeStruct((n, 128), jnp.float32),
                  jax.ShapeDtypeStruct((n, 128), jnp.float32),
                  jax.ShapeDtypeStruct((n, 16), jnp.float32),
                  jax.ShapeDtypeStruct((n, 16), jnp.float32)),
        scratch_types=[
            pltpu.VMEM((sbc, k), jnp.int32),
            pltpu.VMEM((sbc, k), jnp.int32),
            [pltpu.VMEM((k, 16), jnp.float32) for _ in range(nbuf)],
            [pltpu.VMEM((k, 16), jnp.float32) for _ in range(nbuf)],
            [pltpu.VMEM((k, 128), jnp.float32) for _ in range(nbuf)],
            [pltpu.VMEM((k, 128), jnp.float32) for _ in range(nbuf)],
            [pltpu.VMEM((k, 16), jnp.float32) for _ in range(nbuf)],
            pltpu.VMEM_SHARED((n, 128), jnp.float32),
            pltpu.VMEM_SHARED((n, 16), jnp.float32),
            [pltpu.SemaphoreType.DMA for _ in range(nbuf)],
            [pltpu.SemaphoreType.DMA for _ in range(nbuf)],
        ],
    )
    def sc_edge(src_hbm, dst_hbm, as2_hbm, ad2_hbm, xp_hbm,
                on0_hbm, on1_hbm, od0_hbm, od1_hbm,
                src_sb, dst_sb, as2_v, ad2_v, xp_v, msg_v, wb_v,
                accn, accd, sg, ss):
        cid = lax.axis_index("c")
        sid = lax.axis_index("s")

        @plsc.parallel_loop(0, k)
        def _(r):
            for q in range(8):
                msg_v[0][r, pl.ds(16 * q, 16)] = jnp.zeros((16,), jnp.float32)
            wb_v[0][r, :] = jnp.zeros((16,), jnp.float32)

        def init_chunk(j, c):
            ch = sid + ns * j
            @pl.when(ch < n_row_chunks)
            def _():
                base = pl.multiple_of(ch * k, 8)
                pltpu.sync_copy(msg_v[0], accn.at[pl.ds(base, k)])
                pltpu.sync_copy(wb_v[0], accd.at[pl.ds(base, k)])
            return c
        lax.fori_loop(0, (n_row_chunks + ns - 1) // ns, init_chunk, 0)
        plsc.subcore_barrier()

        rbase = cid * (e_per_core // k) + sid * n_edge_chunks

        def issue_gathers(ch, b):
            pltpu.async_copy(as2_hbm.at[src_sb.at[ch]], as2_v[b], sg[b])
            pltpu.async_copy(ad2_hbm.at[dst_sb.at[ch]], ad2_v[b], sg[b])
            pltpu.async_copy(xp_hbm.at[src_sb.at[ch]], xp_v[b], sg[b])

        def wait_gathers(b):
            pltpu.make_async_copy(as2_hbm.at[src_sb.at[0]], as2_v[b],
                                  sg[b]).wait()
            pltpu.make_async_copy(ad2_hbm.at[dst_sb.at[0]], ad2_v[b],
                                  sg[b]).wait()
            pltpu.make_async_copy(xp_hbm.at[src_sb.at[0]], xp_v[b],
                                  sg[b]).wait()

        def wait_scatter(b):
            pltpu.make_async_copy(msg_v[b], accn.at[dst_sb.at[0]],
                                  ss[b]).wait()
            pltpu.make_async_copy(wb_v[b], accd.at[dst_sb.at[0]],
                                  ss[b]).wait()

        for isb in range(nsb):
            pltpu.sync_copy(src_hbm.at[pl.ds(rbase + isb * sbc, sbc)], src_sb)
            pltpu.sync_copy(dst_hbm.at[pl.ds(rbase + isb * sbc, sbc)], dst_sb)
            for b in range(nbuf):
                issue_gathers(b, b)

            @pl.loop(0, sbc + nbuf - 2, step=nbuf)
            def _(i):
                for b in range(nbuf):
                    ch = i + b
                    @pl.when(ch < sbc)
                    def _():
                        wait_gathers(b)
                        @pl.when(ch >= nbuf)
                        def _():
                            wait_scatter(b)

                        @plsc.parallel_loop(0, k, unroll=4)
                        def _(ei):
                            z = as2_v[b][ei] + ad2_v[b][ei]
                            w = jnp.exp(jnp.maximum(z, 0.2 * z))
                            wb_v[b][ei, :] = w
                            for q in range(8):
                                msg_v[b][ei, pl.ds(16 * q, 16)] = (
                                    w * xp_v[b][ei, pl.ds(16 * q, 16)])

                        pltpu.async_copy(msg_v[b], accn.at[dst_sb.at[ch]],
                                         ss[b], add=True)
                        pltpu.async_copy(wb_v[b], accd.at[dst_sb.at[ch]],
                                         ss[b], add=True)
                        @pl.when(ch + nbuf < sbc)
                        def _():
                            issue_gathers(ch + nbuf, b)

            for b in range(nbuf):
                wait_scatter(b)
        plsc.subcore_barrier()

        def out_chunk(j, c):
            ch = sid + ns * j
            @pl.when(ch < n_row_chunks)
            def _():
                base = pl.multiple_of(ch * k, 8)
                pltpu.sync_copy(accn.at[pl.ds(base, k)], msg_v[0])
                pltpu.sync_copy(accd.at[pl.ds(base, k)], wb_v[0])
                @pl.when(cid == 0)
                def _():
                    pltpu.sync_copy(msg_v[0], on0_hbm.at[pl.ds(base, k)])
                    pltpu.sync_copy(wb_v[0], od0_hbm.at[pl.ds(base, k)])
                @pl.when(cid == 1)
                def _():
                    pltpu.sync_copy(msg_v[0], on1_hbm.at[pl.ds(base, k)])
                    pltpu.sync_copy(wb_v[0], od1_hbm.at[pl.ds(base, k)])
            return c
        lax.fori_loop(0, (n_row_chunks + ns - 1) // ns, out_chunk, 0)

    return sc_edge


def kernel(x, W, att_src, att_dst, bias, res_W, ln1_g, ln1_b, ln2_g, ln2_b,
           Wf1, bf1, Wf2, bf2, edge_index):
    n, d = x.shape
    e = edge_index.shape[1]
    dff = Wf1.shape[1]
    rb = 5000
    grid = (n // rb,)

    perm = jnp.asarray(_PERM)
    wp = W[:, perm]
    rwp = res_W[:, perm]
    ast = att_src.reshape(1, d)[:, perm]
    adt = att_dst.reshape(1, d)[:, perm]
    bias_p = bias.reshape(1, d)[:, perm]
    g2p = ln2_g.reshape(1, d)[:, perm]
    b2p = ln2_b.reshape(1, d)[:, perm]
    wf1p = Wf1[perm, :]
    g1 = ln1_g.reshape(1, d)
    b1 = ln1_b.reshape(1, d)
    s16 = jnp.asarray(_S16)
    rt = jnp.asarray(_RT)
    pm = jnp.asarray(_PM)

    xp_t, as2, ad2, sw, res_t = pl.pallas_call(
        _tc_pre_body,
        grid=grid,
        in_specs=[
            _rows(rb, d), _full((1, d)), _full((1, d)), _full((d, d)),
            _full((1, d)), _full((1, d)), _full((d, d)), _full((d, 16)),
        ],
        out_specs=[
            _rows(rb, d), _rows(rb, 16), _rows(rb, 16),
            _rows(rb, 8), _rows(rb, d),
        ],
        out_shape=[
            jax.ShapeDtypeStruct((n, d), jnp.float32),
            jax.ShapeDtypeStruct((n, 16), jnp.float32),
            jax.ShapeDtypeStruct((n, 16), jnp.float32),
            jax.ShapeDtypeStruct((n, 8), jnp.float32),
            jax.ShapeDtypeStruct((n, d), jnp.float32),
        ],
    )(x, g1, b1, wp, ast, adt, rwp, s16)

    on0, on1, od0, od1 = _make_sc_edge_call(n, e)(
        edge_index[0].reshape(e // 40, 40), edge_index[1].reshape(e // 40, 40),
        as2, ad2, xp_t)

    y = pl.pallas_call(
        _tc_post_body,
        grid=grid,
        in_specs=[
            _rows(rb, d), _rows(rb, d), _rows(rb, 16), _rows(rb, 16),
            _rows(rb, d), _rows(rb, 8),
            _rows(rb, d), _full((1, d)),
            _full((1, d)), _full((1, d)), _full((d, dff)), _full((1, dff)),
            _full((dff, d)), _full((1, d)), _full((d, d)), _full((8, d)),
        ],
        out_specs=_rows(rb, d),
        out_shape=jax.ShapeDtypeStruct((n, d), jnp.float32),
    )(on0, on1, od0, od1, xp_t, sw, res_t, bias_p, g2p, b2p, wf1p,
      bf1.reshape(1, dff), Wf2, bf2.reshape(1, d), pm, rt)

    return y


# speedup vs baseline: 1.0186x; 1.0186x over previous
import functools

import numpy as np
import jax
import jax.numpy as jnp
from jax import lax
from jax.experimental import pallas as pl
from jax.experimental.pallas import tpu as pltpu
from jax.experimental.pallas import tpu_sc as plsc

_H = 8
_C = 16
_ACCW = 144

_PERM = np.empty(128, np.int32)
for _k in range(8):
    for _j in range(16):
        _PERM[16 * _k + _j] = (_j % 8) * 16 + (2 * _k + _j // 8)

_PM = np.zeros((128, 128), np.float32)
for _i in range(128):
    _PM[_i, _PERM[_i]] = 1.0

_RT = np.zeros((8, 128), np.float32)
for _j in range(128):
    _RT[_PERM[_j] // 16, _j] = 1.0

_S16 = np.zeros((128, 16), np.float32)
for _j in range(128):
    _S16[_j, _PERM[_j] // 16] = 1.0
    _S16[_j, 8 + _PERM[_j] // 16] = 1.0


def _tc_pre_body(x_ref, g1_ref, b1_ref, wp_ref, ast_ref, adt_ref, rwp_ref,
                 s16_ref, xp_ref, as2_ref, ad2_ref, sw_ref, res_ref):
    xb = x_ref[...]
    mu = jnp.mean(xb, axis=-1, keepdims=True)
    xc = xb - mu
    var = jnp.mean(xc * xc, axis=-1, keepdims=True)
    h = xc / jnp.sqrt(var + 1e-5) * g1_ref[...] + b1_ref[...]
    xp_t = jnp.dot(h, wp_ref[...], preferred_element_type=jnp.float32)
    as2 = jnp.dot(xp_t * ast_ref[...], s16_ref[...],
                  preferred_element_type=jnp.float32)
    ad2 = jnp.dot(xp_t * adt_ref[...], s16_ref[...],
                  preferred_element_type=jnp.float32)
    res_t = jnp.dot(h, rwp_ref[...], preferred_element_type=jnp.float32)
    z = as2[:, :8] + ad2[:, :8]
    w = jnp.exp(jnp.maximum(z, 0.2 * z))
    xp_ref[...] = xp_t
    as2_ref[...] = as2
    ad2_ref[...] = ad2
    sw_ref[...] = w
    res_ref[...] = res_t


def _tc_post_body(o0_ref, o1_ref, d0_ref, d1_ref, xp_ref, sw_ref, res_ref,
                  bp_ref, g2_ref, b2_ref, wf1_ref, bf1_ref, wf2_ref, bf2_ref,
                  pm_ref, rt_ref, y_ref):
    sw = sw_ref[...]
    swrep = jnp.dot(sw, rt_ref[...], preferred_element_type=jnp.float32)
    num = o0_ref[...] + o1_ref[...] + xp_ref[...] * swrep
    den = (d0_ref[...] + d1_ref[...])[:, :8] + sw
    den_rep = jnp.dot(den, rt_ref[...], preferred_element_type=jnp.float32)
    g = num / (den_rep + 1e-16) + res_ref[...] + bp_ref[...]
    mu = jnp.mean(g, axis=-1, keepdims=True)
    gc = g - mu
    var = jnp.mean(gc * gc, axis=-1, keepdims=True)
    h2 = gc / jnp.sqrt(var + 1e-5) * g2_ref[...] + b2_ref[...]
    u = jnp.dot(h2, wf1_ref[...], preferred_element_type=jnp.float32)
    u = u + bf1_ref[...]
    ff = 0.5 * u * (1.0 + jnp.tanh(np.sqrt(2.0 / np.pi)
                                   * (u + 0.044715 * u * u * u)))
    y = jnp.dot(h2, pm_ref[...], preferred_element_type=jnp.float32)
    y = y + jnp.dot(ff, wf2_ref[...], preferred_element_type=jnp.float32)
    y_ref[...] = y + bf2_ref[...]


def _full(shape):
    return pl.BlockSpec(shape, lambda i: (0,) * len(shape))


def _rows(rb, w):
    return pl.BlockSpec((rb, w), lambda i: (i, 0))


def _make_sc_edge_call(n, e):
    nc, ns = 2, 16
    k = 40
    nbuf = 3
    sbc = 50
    e_per_core = e // nc
    e_per_tile = e_per_core // ns
    n_edge_chunks = e_per_tile // k
    nsb = n_edge_chunks // sbc
    assert e_per_tile % k == 0 and n % k == 0 and n_edge_chunks % sbc == 0
    n_row_chunks = n // k
    mesh = plsc.VectorSubcoreMesh(core_axis_name="c", subcore_axis_name="s")

    @functools.partial(
        pl.kernel,
        mesh=mesh,
        compiler_params=pltpu.CompilerParams(use_tc_tiling_on_sc=False),
        out_type=(jax.ShapeDtypeStruct((n, 128), jnp.float32),
                  jax.ShapeDtypeStruct((n, 128), jnp.float32),
                  jax.ShapeDtypeStruct((n, 16), jnp.float32),
                  jax.ShapeDtypeStruct((n, 16), jnp.float32)),
        scratch_types=[
            pltpu.VMEM((sbc, k), jnp.int32),
            pltpu.VMEM((sbc, k), jnp.int32),
            [pltpu.VMEM((k, 16), jnp.float32) for _ in range(nbuf)],
            [pltpu.VMEM((k, 16), jnp.float32) for _ in range(nbuf)],
            [pltpu.VMEM((k, 128), jnp.float32) for _ in range(nbuf)],
            [pltpu.VMEM((k, 128), jnp.float32) for _ in range(nbuf)],
            [pltpu.VMEM((k, 16), jnp.float32) for _ in range(nbuf)],
            pltpu.VMEM_SHARED((n, 128), jnp.float32),
            pltpu.VMEM_SHARED((n, 16), jnp.float32),
            [pltpu.SemaphoreType.DMA for _ in range(nbuf)],
            [pltpu.SemaphoreType.DMA for _ in range(nbuf)],
        ],
    )
    def sc_edge(src_hbm, dst_hbm, as2_hbm, ad2_hbm, xp_hbm,
                on0_hbm, on1_hbm, od0_hbm, od1_hbm,
                src_sb, dst_sb, as2_v, ad2_v, xp_v, msg_v, wb_v,
                accn, accd, sg, ss):
        cid = lax.axis_index("c")
        sid = lax.axis_index("s")

        @plsc.parallel_loop(0, k)
        def _(r):
            for q in range(8):
                msg_v[0][r, pl.ds(16 * q, 16)] = jnp.zeros((16,), jnp.float32)
            wb_v[0][r, :] = jnp.zeros((16,), jnp.float32)

        def init_chunk(j, c):
            ch = sid + ns * j
            @pl.when(ch < n_row_chunks)
            def _():
                base = pl.multiple_of(ch * k, 8)
                pltpu.sync_copy(msg_v[0], accn.at[pl.ds(base, k)])
                pltpu.sync_copy(wb_v[0], accd.at[pl.ds(base, k)])
            return c
        lax.fori_loop(0, (n_row_chunks + ns - 1) // ns, init_chunk, 0)
        plsc.subcore_barrier()

        rbase = cid * (e_per_core // k) + sid * n_edge_chunks

        def issue_gathers(ch, b):
            pltpu.async_copy(as2_hbm.at[src_sb.at[ch]], as2_v[b], sg[b])
            pltpu.async_copy(ad2_hbm.at[dst_sb.at[ch]], ad2_v[b], sg[b])
            pltpu.async_copy(xp_hbm.at[src_sb.at[ch]], xp_v[b], sg[b])

        def wait_gathers(b):
            pltpu.make_async_copy(as2_hbm.at[src_sb.at[0]], as2_v[b],
                                  sg[b]).wait()
            pltpu.make_async_copy(ad2_hbm.at[dst_sb.at[0]], ad2_v[b],
                                  sg[b]).wait()
            pltpu.make_async_copy(xp_hbm.at[src_sb.at[0]], xp_v[b],
                                  sg[b]).wait()

        def wait_scatter(b):
            pltpu.make_async_copy(msg_v[b], accn.at[dst_sb.at[0]],
                                  ss[b]).wait()
            pltpu.make_async_copy(wb_v[b], accd.at[dst_sb.at[0]],
                                  ss[b]).wait()

        for isb in range(nsb):
            pltpu.sync_copy(src_hbm.at[pl.ds(rbase + isb * sbc, sbc)], src_sb)
            pltpu.sync_copy(dst_hbm.at[pl.ds(rbase + isb * sbc, sbc)], dst_sb)
            for b in range(nbuf):
                issue_gathers(b, b)

            @pl.loop(0, sbc + nbuf - 2, step=nbuf)
            def _(i):
                for b in range(nbuf):
                    ch = i + b
                    @pl.when(ch < sbc)
                    def _():
                        wait_gathers(b)
                        @pl.when(ch >= nbuf)
                        def _():
                            wait_scatter(b)

                        @plsc.parallel_loop(0, k, unroll=4)
                        def _(ei):
                            z = as2_v[b][ei] + ad2_v[b][ei]
                            w = jnp.exp(jnp.maximum(z, 0.2 * z))
                            wb_v[b][ei, :] = w
                            for q in range(8):
                                msg_v[b][ei, pl.ds(16 * q, 16)] = (
                                    w * xp_v[b][ei, pl.ds(16 * q, 16)])

                        pltpu.async_copy(msg_v[b], accn.at[dst_sb.at[ch]],
                                         ss[b], add=True)
                        pltpu.async_copy(wb_v[b], accd.at[dst_sb.at[ch]],
                                         ss[b], add=True)
                        @pl.when(ch + nbuf < sbc)
                        def _():
                            issue_gathers(ch + nbuf, b)

            for b in range(nbuf):
                wait_scatter(b)
        plsc.subcore_barrier()

        def out_chunk(j, c):
            ch = sid + ns * j
            @pl.when(ch < n_row_chunks)
            def _():
                base = pl.multiple_of(ch * k, 8)
                pltpu.sync_copy(accn.at[pl.ds(base, k)], msg_v[0])
                pltpu.sync_copy(accd.at[pl.ds(base, k)], wb_v[0])
                @pl.when(cid == 0)
                def _():
                    pltpu.sync_copy(msg_v[0], on0_hbm.at[pl.ds(base, k)])
                    pltpu.sync_copy(wb_v[0], od0_hbm.at[pl.ds(base, k)])
                @pl.when(cid == 1)
                def _():
                    pltpu.sync_copy(msg_v[0], on1_hbm.at[pl.ds(base, k)])
                    pltpu.sync_copy(wb_v[0], od1_hbm.at[pl.ds(base, k)])
            return c
        lax.fori_loop(0, (n_row_chunks + ns - 1) // ns, out_chunk, 0)

    return sc_edge


def kernel(x, W, att_src, att_dst, bias, res_W, ln1_g, ln1_b, ln2_g, ln2_b,
           Wf1, bf1, Wf2, bf2, edge_index):
    n, d = x.shape
    e = edge_index.shape[1]
    dff = Wf1.shape[1]
    rb = 2000
    grid = (n // rb,)

    perm = jnp.asarray(_PERM)
    wp = W[:, perm]
    rwp = res_W[:, perm]
    ast = att_src.reshape(1, d)[:, perm]
    adt = att_dst.reshape(1, d)[:, perm]
    bias_p = bias.reshape(1, d)[:, perm]
    g2p = ln2_g.reshape(1, d)[:, perm]
    b2p = ln2_b.reshape(1, d)[:, perm]
    wf1p = Wf1[perm, :]
    g1 = ln1_g.reshape(1, d)
    b1 = ln1_b.reshape(1, d)
    s16 = jnp.asarray(_S16)
    rt = jnp.asarray(_RT)
    pm = jnp.asarray(_PM)

    xp_t, as2, ad2, sw, res_t = pl.pallas_call(
        _tc_pre_body,
        grid=grid,
        in_specs=[
            _rows(rb, d), _full((1, d)), _full((1, d)), _full((d, d)),
            _full((1, d)), _full((1, d)), _full((d, d)), _full((d, 16)),
        ],
        out_specs=[
            _rows(rb, d), _rows(rb, 16), _rows(rb, 16),
            _rows(rb, 8), _rows(rb, d),
        ],
        out_shape=[
            jax.ShapeDtypeStruct((n, d), jnp.float32),
            jax.ShapeDtypeStruct((n, 16), jnp.float32),
            jax.ShapeDtypeStruct((n, 16), jnp.float32),
            jax.ShapeDtypeStruct((n, 8), jnp.float32),
            jax.ShapeDtypeStruct((n, d), jnp.float32),
        ],
    )(x, g1, b1, wp, ast, adt, rwp, s16)

    on0, on1, od0, od1 = _make_sc_edge_call(n, e)(
        edge_index[0].reshape(e // 40, 40), edge_index[1].reshape(e // 40, 40),
        as2, ad2, xp_t)

    y = pl.pallas_call(
        _tc_post_body,
        grid=grid,
        in_specs=[
            _rows(rb, d), _rows(rb, d), _rows(rb, 16), _rows(rb, 16),
            _rows(rb, d), _rows(rb, 8),
            _rows(rb, d), _full((1, d)),
            _full((1, d)), _full((1, d)), _full((d, dff)), _full((1, dff)),
            _full((dff, d)), _full((1, d)), _full((d, d)), _full((8, d)),
        ],
        out_specs=_rows(rb, d),
        out_shape=jax.ShapeDtypeStruct((n, d), jnp.float32),
    )(on0, on1, od0, od1, xp_t, sw, res_t, bias_p, g2p, b2p, wf1p,
      bf1.reshape(1, dff), Wf2, bf2.reshape(1, d), pm, rt)

    return y
